# baseline (speedup 1.0000x reference)
import jax
import jax.numpy as jnp
from jax.experimental import pallas as pl

B = 50
NG = 200
K = 4
EPS = 1e-5
KP1 = 160
KP2 = 128
GSTEP = 10
HP = jax.lax.Precision.HIGHEST
F32 = jnp.float32
BF16 = jnp.bfloat16
INTERPRET = False


def _dot(a, b):
    return jnp.dot(a, b, precision=HP, preferred_element_type=F32)


def _bdot(a, b):
    return jnp.dot(a.astype(BF16), b.astype(BF16), preferred_element_type=F32)


def _row(col, n):
    ir = jax.lax.broadcasted_iota(jnp.int32, (n, n), 0)
    ic = jax.lax.broadcasted_iota(jnp.int32, (n, n), 1)
    return jnp.sum(jnp.where(ir == ic, col, 0.0), axis=0, keepdims=True)


def _split3(x):
    hi = x.astype(BF16)
    r = x - hi.astype(F32)
    mid = r.astype(BF16)
    lo = (r - mid.astype(F32)).astype(BF16)
    return hi, mid, lo


def _gather(oh, parts):
    hi, mid, lo = parts
    ohb = oh.astype(BF16)
    return (jnp.dot(ohb, hi, preferred_element_type=F32)
            + jnp.dot(ohb, mid, preferred_element_type=F32)) \
        + jnp.dot(ohb, lo, preferred_element_type=F32)


def _k1(pos_ref, x_ref, wcat_ref, b1_ref, nbr_ref, nsum_ref, st_ref):
    b = pl.program_id(0)
    ir = jax.lax.broadcasted_iota(jnp.int32, (NG, NG), 0)
    ic = jax.lax.broadcasted_iota(jnp.int32, (NG, NG), 1)
    wcat = wcat_ref[...]
    b1 = b1_ref[...]
    s1t = jnp.zeros((1, 128), F32)
    s2t = jnp.zeros((1, 128), F32)

    for gi in range(GSTEP):
        p = pos_ref[gi]
        x = x_ref[gi]
        p2 = jnp.sum(p * p, axis=1, keepdims=True)
        pb = p.astype(BF16)
        g = jax.lax.dot_general(pb, pb, (((1,), (1,)), ((), ())),
                                preferred_element_type=F32)
        d2 = p2 + _row(p2, NG) - 2.0 * g
        d2 = jnp.where(ir == ic, d2 + 1e9, d2)

        nbrs = []
        d = d2
        for _ in range(K):
            mn = jnp.min(d, axis=1, keepdims=True)
            idx = jnp.min(jnp.where(d == mn, ic, NG), axis=1, keepdims=True)
            nbrs.append(idx)
            d = jnp.where(ic == idx, 1e9, d)

        oh = jnp.concatenate(
            [(ic == nbrs[k]).astype(F32) for k in range(K)], axis=0)
        xj = _gather(oh, _split3(x))
        xrep = jnp.concatenate([x] * K, axis=0)
        cat = jnp.concatenate([xrep, xj - xrep], axis=1)
        h = jnp.maximum(_bdot(cat, wcat) + b1, 0.0)
        acc = ((h[0:NG] + h[NG:2 * NG]) + h[2 * NG:3 * NG]) + h[3 * NG:4 * NG]
        s1t = s1t + jnp.sum(h, axis=0, keepdims=True)
        s2t = s2t + jnp.sum(h * h, axis=0, keepdims=True)

        nsum_ref[gi] = acc
        nbr_ref[gi] = jnp.concatenate(
            nbrs + [jnp.zeros((NG, 4), jnp.int32)], axis=1)

    @pl.when(b == 0)
    def _():
        st_ref[...] = jnp.zeros_like(st_ref)

    gcb = jnp.zeros((1, 128), F32) + float(NG * K * GSTEP)
    upd = jnp.concatenate([s1t, s2t, gcb, jnp.zeros((5, 128), F32)], axis=0)
    st_ref[...] = st_ref[...] + upd


def _mk_pool_conv(n, kkeep, fin, fout):
    def body(nsum_ref, cnt_ref, nbr_ref, m_ref, st_ref, g_ref, be_ref,
             pw_ref, wcat_ref, b2_ref,
             nsum2_ref, cnt2_ref, nbr2_ref, m2_ref, st2_ref):
        b = pl.program_id(0)
        s1 = st_ref[0:1]
        s2 = st_ref[1:2]
        gc = jnp.maximum(st_ref[2:3], 1.0)
        mu = s1 / gc
        var = s2 / gc - mu * mu
        a = g_ref[...] / jnp.sqrt(var + EPS)
        dsh = be_ref[...] - mu * a
        w = pw_ref[...]
        nw = jnp.sqrt(jnp.sum(w * w, axis=0, keepdims=True)[:, 0:1])
        wcat = wcat_ref[...]
        b2 = b2_ref[...]
        ir = jax.lax.broadcasted_iota(jnp.int32, (n, n), 0)
        ic = jax.lax.broadcasted_iota(jnp.int32, (n, n), 1)
        ltri = (ic <= ir).astype(F32)
        ick = jax.lax.broadcasted_iota(jnp.int32, (kkeep, kkeep), 1)
        s1t = jnp.zeros((1, fout), F32)
        s2t = jnp.zeros((1, fout), F32)
        gct = jnp.zeros((1, 1), F32)

        for gi in range(GSTEP):
            cnt = cnt_ref[gi][:, 0:1]
            h = a * (nsum_ref[gi] / jnp.maximum(cnt, 1.0)) \
                + dsh * (cnt > 0.0).astype(F32)
            y = _bdot(h, w)[:, 0:1]
            sc = jnp.tanh(y / nw)

            srow = _row(sc, n)
            rmat = (srow > sc) | ((srow == sc) & (ic < ir))
            rank = jnp.sum(rmat.astype(F32), axis=1, keepdims=True)
            keep = (rank < float(kkeep)).astype(F32)
            pref = _bdot(ltri, keep)
            newid = jnp.where(keep > 0, pref - 1.0, -1.0)
            nrow = _row(newid, n).astype(jnp.int32)
            ck = (jax.lax.broadcasted_iota(jnp.int32, (kkeep, n), 0)
                  == nrow).astype(F32)

            x2 = _gather(ck, _split3(h * sc))

            oho = jnp.concatenate(
                [(ic == nbr_ref[gi][:, kk:kk + 1]).astype(F32)
                 for kk in range(K)], axis=0)
            nid_all = _bdot(oho, newid)
            mall_old = jnp.concatenate(
                [m_ref[gi][:, kk:kk + 1] for kk in range(K)], axis=0)
            mpre_all = mall_old * (nid_all >= 0.0).astype(F32)
            nidn = jnp.concatenate(
                [nid_all[kk * n:(kk + 1) * n] for kk in range(K)], axis=1)
            mpre = jnp.concatenate(
                [mpre_all[kk * n:(kk + 1) * n] for kk in range(K)], axis=1)
            nm8 = _bdot(ck, jnp.concatenate([nidn, mpre], axis=1))
            nbr2 = nm8[:, 0:4]
            m2 = nm8[:, 4:8]
            nbr2i = jnp.maximum(nbr2, 0.0).astype(jnp.int32)

            oh = jnp.concatenate(
                [(ick == nbr2i[:, kk:kk + 1]).astype(F32) for kk in range(K)],
                axis=0)
            xj = _gather(oh, _split3(x2))
            xrep = jnp.concatenate([x2] * K, axis=0)
            cat = jnp.concatenate([xrep, xj - xrep], axis=1)
            hh = jnp.maximum(_bdot(cat, wcat) + b2, 0.0)
            mall = jnp.concatenate(
                [m2[:, kk:kk + 1] for kk in range(K)], axis=0)
            hm = hh * mall
            kk_ = kkeep
            acc = ((hm[0:kk_] + hm[kk_:2 * kk_]) + hm[2 * kk_:3 * kk_]) \
                + hm[3 * kk_:4 * kk_]
            cnt2 = ((m2[:, 0:1] + m2[:, 1:2]) + m2[:, 2:3]) + m2[:, 3:4]
            s1t = s1t + jnp.sum(hm, axis=0, keepdims=True)
            s2t = s2t + jnp.sum(hm * hh, axis=0, keepdims=True)
            gct = gct + jnp.sum(mall, axis=0, keepdims=True)

            nsum2_ref[gi] = acc
            cnt2_ref[gi] = jnp.concatenate([cnt2] * 8, axis=1)
            nbr2_ref[gi] = jnp.concatenate(
                [nbr2i, jnp.zeros((kkeep, 4), jnp.int32)], axis=1)
            m2_ref[gi] = jnp.concatenate(
                [m2, jnp.zeros((kkeep, 4), F32)], axis=1)

        @pl.when(b == 0)
        def _():
            st2_ref[...] = jnp.zeros_like(st2_ref)

        gcb = jnp.zeros((1, fout), F32) + gct
        upd = jnp.concatenate(
            [s1t, s2t, gcb, jnp.zeros((5, fout), F32)], axis=0)
        st2_ref[...] = st2_ref[...] + upd

    return body


def _k4(nsum_ref, cnt_ref, st_ref, g_ref, be_ref, hg_ref):
    s1 = st_ref[0:1]
    s2 = st_ref[1:2]
    gc = jnp.maximum(st_ref[2:3], 1.0)
    mu = s1 / gc
    var = s2 / gc - mu * mu
    a = g_ref[...] / jnp.sqrt(var + EPS)
    dsh = be_ref[...] - mu * a
    for gi in range(GSTEP):
        cnt = cnt_ref[gi][:, 0:1]
        h = a * (nsum_ref[gi] / jnp.maximum(cnt, 1.0)) \
            + dsh * (cnt > 0.0).astype(F32)
        hg_ref[gi] = jnp.sum(h, axis=0, keepdims=True) / float(KP2)


def _k5(hg_ref, wc1_ref, bc1_ref, g1_ref, be1_ref,
        wc2_ref, bc2_ref, g2_ref, be2_ref, o_ref):
    hg = hg_ref[...]
    rows = (jax.lax.broadcasted_iota(jnp.int32, (56, 1), 0) < B).astype(F32)
    z = jnp.maximum(_bdot(hg, wc1_ref[...]) + bc1_ref[...], 0.0)
    z = z * rows
    mu = jnp.sum(z, axis=0, keepdims=True) / float(B)
    zc = z - mu
    var = jnp.sum(zc * zc * rows, axis=0, keepdims=True) / float(B)
    z = zc / jnp.sqrt(var + EPS) * g1_ref[...] + be1_ref[...]
    z2 = jnp.maximum(_bdot(z, wc2_ref[...]) + bc2_ref[...], 0.0)
    z2m = z2 * rows
    mu2 = jnp.sum(z2m, axis=0, keepdims=True) / float(B)
    zc2 = z2 - mu2
    var2 = jnp.sum(zc2 * zc2 * rows, axis=0, keepdims=True) / float(B)
    o_ref[...] = jax.nn.sigmoid(
        zc2 / jnp.sqrt(var2 + EPS) * g2_ref[...] + be2_ref[...])


def _gspec(shape):
    return pl.BlockSpec((GSTEP,) + shape, lambda b: (b, 0, 0))


def _cspec(shape):
    return pl.BlockSpec(shape, lambda b: (0,) * len(shape))


def kernel(x, pos, batch, W1, b1, g1, be1, pw1, W2, b2, g2, be2, pw2,
           W3, b3, g3, be3, Wc1, bc1, gc1, bec1, Wc2, bc2, gc2, bec2):
    f32 = jnp.float32
    xg = jnp.pad(x, ((0, 0), (0, 11))).reshape(B, NG, 64)
    pg = jnp.pad(pos, ((0, 0), (0, 5))).reshape(B, NG, 8)
    wcat1 = jnp.concatenate([
        jnp.pad(W1[:53], ((0, 11), (0, 0))),
        jnp.pad(W1[53:], ((0, 11), (0, 0)))], axis=0)

    r2 = lambda v: v.reshape(1, -1).astype(f32)
    pcol = lambda v: jnp.pad(v.reshape(-1, 1), ((0, 0), (0, 127))).astype(f32)

    nbr1, nsum1, st1 = pl.pallas_call(
        _k1,
        grid=(B // GSTEP,),
        in_specs=[_gspec((NG, 8)), _gspec((NG, 64)),
                  _cspec((128, 128)), _cspec((1, 128))],
        out_specs=[_gspec((NG, 8)), _gspec((NG, 128)), _cspec((8, 128))],
        out_shape=[jax.ShapeDtypeStruct((B, NG, 8), jnp.int32),
                   jax.ShapeDtypeStruct((B, NG, 128), f32),
                   jax.ShapeDtypeStruct((8, 128), f32)],
        interpret=INTERPRET,
    )(pg, xg, wcat1, r2(b1))

    cnt1 = jnp.full((B, NG, 8), 4.0, f32)
    m1 = jnp.ones((B, NG, 8), f32)

    body2 = _mk_pool_conv(NG, KP1, 128, 256)
    nsum2, cnt2, nbr2, m2, st2 = pl.pallas_call(
        body2,
        grid=(B // GSTEP,),
        in_specs=[_gspec((NG, 128)), _gspec((NG, 8)), _gspec((NG, 8)),
                  _gspec((NG, 8)), _cspec((8, 128)), _cspec((1, 128)),
                  _cspec((1, 128)), _cspec((128, 128)),
                  _cspec((256, 256)), _cspec((1, 256))],
        out_specs=[_gspec((KP1, 256)), _gspec((KP1, 8)), _gspec((KP1, 8)),
                   _gspec((KP1, 8)), _cspec((8, 256))],
        out_shape=[jax.ShapeDtypeStruct((B, KP1, 256), f32),
                   jax.ShapeDtypeStruct((B, KP1, 8), f32),
                   jax.ShapeDtypeStruct((B, KP1, 8), jnp.int32),
                   jax.ShapeDtypeStruct((B, KP1, 8), f32),
                   jax.ShapeDtypeStruct((8, 256), f32)],
        interpret=INTERPRET,
    )(nsum1, cnt1, nbr1, m1, st1, r2(g1), r2(be1), pcol(pw1), W2, r2(b2))

    body3 = _mk_pool_conv(KP1, KP2, 256, 512)
    nsum3, cnt3, nbr3, m3, st3 = pl.pallas_call(
        body3,
        grid=(B // GSTEP,),
        in_specs=[_gspec((KP1, 256)), _gspec((KP1, 8)), _gspec((KP1, 8)),
                  _gspec((KP1, 8)), _cspec((8, 256)), _cspec((1, 256)),
                  _cspec((1, 256)), _cspec((256, 128)),
                  _cspec((512, 512)), _cspec((1, 512))],
        out_specs=[_gspec((KP2, 512)), _gspec((KP2, 8)), _gspec((KP2, 8)),
                   _gspec((KP2, 8)), _cspec((8, 512))],
        out_shape=[jax.ShapeDtypeStruct((B, KP2, 512), f32),
                   jax.ShapeDtypeStruct((B, KP2, 8), f32),
                   jax.ShapeDtypeStruct((B, KP2, 8), jnp.int32),
                   jax.ShapeDtypeStruct((B, KP2, 8), f32),
                   jax.ShapeDtypeStruct((8, 512), f32)],
        interpret=INTERPRET,
    )(nsum2, cnt2, nbr2, m2, st2, r2(g2), r2(be2), pcol(pw2), W3, r2(b3))
    del nbr3, m3

    hg = pl.pallas_call(
        _k4,
        grid=(B // GSTEP,),
        in_specs=[_gspec((KP2, 512)), _gspec((KP2, 8)), _cspec((8, 512)),
                  _cspec((1, 512)), _cspec((1, 512))],
        out_specs=[_gspec((1, 512))],
        out_shape=[jax.ShapeDtypeStruct((B, 1, 512), f32)],
        interpret=INTERPRET,
    )(nsum3, cnt3, st3, r2(g3), r2(be3))[0]

    hgp = jnp.concatenate(
        [hg.reshape(B, 512), jnp.zeros((6, 512), f32)], axis=0)
    wc2p = jnp.pad(Wc2, ((0, 0), (0, 127)))
    pad1 = lambda v: jnp.pad(v.reshape(1, -1), ((0, 0), (0, 127))).astype(f32)

    out = pl.pallas_call(
        _k5,
        in_specs=[pl.BlockSpec((56, 512), lambda: (0, 0)),
                  pl.BlockSpec((512, 512), lambda: (0, 0)),
                  pl.BlockSpec((1, 512), lambda: (0, 0)),
                  pl.BlockSpec((1, 512), lambda: (0, 0)),
                  pl.BlockSpec((1, 512), lambda: (0, 0)),
                  pl.BlockSpec((512, 128), lambda: (0, 0)),
                  pl.BlockSpec((1, 128), lambda: (0, 0)),
                  pl.BlockSpec((1, 128), lambda: (0, 0)),
                  pl.BlockSpec((1, 128), lambda: (0, 0))],
        out_specs=pl.BlockSpec((56, 128), lambda: (0, 0)),
        out_shape=jax.ShapeDtypeStruct((56, 128), f32),
        interpret=INTERPRET,
    )(hgp, Wc1, r2(bc1), r2(gc1), r2(bec1), wc2p, pad1(bc2),
      pad1(gc2), pad1(bec2))

    return out[:B, 0]


# speedup vs baseline: 1.0332x; 1.0332x over previous
import jax
import jax.numpy as jnp
from jax.experimental import pallas as pl

B = 50
NG = 200
K = 4
EPS = 1e-5
KP1 = 160
KP2 = 128
GSTEP = 5
HP = jax.lax.Precision.HIGHEST
F32 = jnp.float32
BF16 = jnp.bfloat16
INTERPRET = False


def _dot(a, b):
    return jnp.dot(a, b, precision=HP, preferred_element_type=F32)


def _bdot(a, b):
    return jnp.dot(a.astype(BF16), b.astype(BF16), preferred_element_type=F32)


def _row(col, n):
    ir = jax.lax.broadcasted_iota(jnp.int32, (n, n), 0)
    ic = jax.lax.broadcasted_iota(jnp.int32, (n, n), 1)
    return jnp.sum(jnp.where(ir == ic, col, 0.0), axis=0, keepdims=True)


def _split3(x):
    hi = x.astype(BF16)
    r = x - hi.astype(F32)
    mid = r.astype(BF16)
    lo = (r - mid.astype(F32)).astype(BF16)
    return hi, mid, lo


def _gather(oh, parts):
    hi, mid, lo = parts
    ohb = oh.astype(BF16)
    return (jnp.dot(ohb, hi, preferred_element_type=F32)
            + jnp.dot(ohb, mid, preferred_element_type=F32)) \
        + jnp.dot(ohb, lo, preferred_element_type=F32)


def _k1(pos_ref, x_ref, wcat_ref, b1_ref, nbr_ref, nsum_ref, st_ref):
    b = pl.program_id(0)
    ir = jax.lax.broadcasted_iota(jnp.int32, (NG, NG), 0)
    ic = jax.lax.broadcasted_iota(jnp.int32, (NG, NG), 1)
    wcat = wcat_ref[...]
    b1 = b1_ref[...]
    s1t = jnp.zeros((1, 128), F32)
    s2t = jnp.zeros((1, 128), F32)

    for gi in range(GSTEP):
        p = pos_ref[gi]
        x = x_ref[gi]
        p2 = jnp.sum(p * p, axis=1, keepdims=True)
        pb = p.astype(BF16)
        g = jax.lax.dot_general(pb, pb, (((1,), (1,)), ((), ())),
                                preferred_element_type=F32)
        d2 = p2 + _row(p2, NG) - 2.0 * g
        d2 = jnp.where(ir == ic, d2 + 1e9, d2)

        nbrs = []
        d = d2
        for _ in range(K):
            mn = jnp.min(d, axis=1, keepdims=True)
            idx = jnp.min(jnp.where(d == mn, ic, NG), axis=1, keepdims=True)
            nbrs.append(idx)
            d = jnp.where(ic == idx, 1e9, d)

        oh = jnp.concatenate(
            [(ic == nbrs[k]).astype(F32) for k in range(K)], axis=0)
        xj = _gather(oh, _split3(x))
        xrep = jnp.concatenate([x] * K, axis=0)
        cat = jnp.concatenate([xrep, xj - xrep], axis=1)
        h = jnp.maximum(_bdot(cat, wcat) + b1, 0.0)
        acc = ((h[0:NG] + h[NG:2 * NG]) + h[2 * NG:3 * NG]) + h[3 * NG:4 * NG]
        s1t = s1t + jnp.sum(h, axis=0, keepdims=True)
        s2t = s2t + jnp.sum(h * h, axis=0, keepdims=True)

        nsum_ref[gi] = acc
        nbr_ref[gi] = jnp.concatenate(
            nbrs + [jnp.zeros((NG, 4), jnp.int32)], axis=1)

    @pl.when(b == 0)
    def _():
        st_ref[...] = jnp.zeros_like(st_ref)

    gcb = jnp.zeros((1, 128), F32) + float(NG * K * GSTEP)
    upd = jnp.concatenate([s1t, s2t, gcb, jnp.zeros((5, 128), F32)], axis=0)
    st_ref[...] = st_ref[...] + upd


def _mk_pool_conv(n, kkeep, fin, fout):
    def body(nsum_ref, cnt_ref, nbr_ref, m_ref, st_ref, g_ref, be_ref,
             pw_ref, wcat_ref, b2_ref,
             nsum2_ref, cnt2_ref, nbr2_ref, m2_ref, st2_ref):
        b = pl.program_id(0)
        s1 = st_ref[0:1]
        s2 = st_ref[1:2]
        gc = jnp.maximum(st_ref[2:3], 1.0)
        mu = s1 / gc
        var = s2 / gc - mu * mu
        a = g_ref[...] / jnp.sqrt(var + EPS)
        dsh = be_ref[...] - mu * a
        w = pw_ref[...]
        nw = jnp.sqrt(jnp.sum(w * w, axis=0, keepdims=True)[:, 0:1])
        wcat = wcat_ref[...]
        b2 = b2_ref[...]
        ir = jax.lax.broadcasted_iota(jnp.int32, (n, n), 0)
        ic = jax.lax.broadcasted_iota(jnp.int32, (n, n), 1)
        ltri = (ic <= ir).astype(F32)
        ick = jax.lax.broadcasted_iota(jnp.int32, (kkeep, kkeep), 1)
        s1t = jnp.zeros((1, fout), F32)
        s2t = jnp.zeros((1, fout), F32)
        gct = jnp.zeros((1, 1), F32)

        for gi in range(GSTEP):
            cnt = cnt_ref[gi][:, 0:1]
            h = a * (nsum_ref[gi] / jnp.maximum(cnt, 1.0)) \
                + dsh * (cnt > 0.0).astype(F32)
            y = _bdot(h, w)[:, 0:1]
            sc = jnp.tanh(y / nw)

            srow = _row(sc, n)
            rmat = (srow > sc) | ((srow == sc) & (ic < ir))
            rank = jnp.sum(rmat.astype(F32), axis=1, keepdims=True)
            keep = (rank < float(kkeep)).astype(F32)
            pref = _bdot(ltri, keep)
            newid = jnp.where(keep > 0, pref - 1.0, -1.0)
            nrow = _row(newid, n).astype(jnp.int32)
            ck = (jax.lax.broadcasted_iota(jnp.int32, (kkeep, n), 0)
                  == nrow).astype(F32)

            x2 = _gather(ck, _split3(h * sc))

            oho = jnp.concatenate(
                [(ic == nbr_ref[gi][:, kk:kk + 1]).astype(F32)
                 for kk in range(K)], axis=0)
            nid_all = _bdot(oho, newid)
            mall_old = jnp.concatenate(
                [m_ref[gi][:, kk:kk + 1] for kk in range(K)], axis=0)
            mpre_all = mall_old * (nid_all >= 0.0).astype(F32)
            nidn = jnp.concatenate(
                [nid_all[kk * n:(kk + 1) * n] for kk in range(K)], axis=1)
            mpre = jnp.concatenate(
                [mpre_all[kk * n:(kk + 1) * n] for kk in range(K)], axis=1)
            nm8 = _bdot(ck, jnp.concatenate([nidn, mpre], axis=1))
            nbr2 = nm8[:, 0:4]
            m2 = nm8[:, 4:8]
            nbr2i = jnp.maximum(nbr2, 0.0).astype(jnp.int32)

            oh = jnp.concatenate(
                [(ick == nbr2i[:, kk:kk + 1]).astype(F32) for kk in range(K)],
                axis=0)
            xj = _gather(oh, _split3(x2))
            xrep = jnp.concatenate([x2] * K, axis=0)
            cat = jnp.concatenate([xrep, xj - xrep], axis=1)
            hh = jnp.maximum(_bdot(cat, wcat) + b2, 0.0)
            mall = jnp.concatenate(
                [m2[:, kk:kk + 1] for kk in range(K)], axis=0)
            hm = hh * mall
            kk_ = kkeep
            acc = ((hm[0:kk_] + hm[kk_:2 * kk_]) + hm[2 * kk_:3 * kk_]) \
                + hm[3 * kk_:4 * kk_]
            cnt2 = ((m2[:, 0:1] + m2[:, 1:2]) + m2[:, 2:3]) + m2[:, 3:4]
            s1t = s1t + jnp.sum(hm, axis=0, keepdims=True)
            s2t = s2t + jnp.sum(hm * hh, axis=0, keepdims=True)
            gct = gct + jnp.sum(mall, axis=0, keepdims=True)

            nsum2_ref[gi] = acc
            cnt2_ref[gi] = jnp.concatenate([cnt2] * 8, axis=1)
            nbr2_ref[gi] = jnp.concatenate(
                [nbr2i, jnp.zeros((kkeep, 4), jnp.int32)], axis=1)
            m2_ref[gi] = jnp.concatenate(
                [m2, jnp.zeros((kkeep, 4), F32)], axis=1)

        @pl.when(b == 0)
        def _():
            st2_ref[...] = jnp.zeros_like(st2_ref)

        gcb = jnp.zeros((1, fout), F32) + gct
        upd = jnp.concatenate(
            [s1t, s2t, gcb, jnp.zeros((5, fout), F32)], axis=0)
        st2_ref[...] = st2_ref[...] + upd

    return body


def _k45(nsum_ref, cnt_ref, st_ref, g_ref, be_ref,
         wc1_ref, bc1_ref, g1_ref, be1_ref,
         wc2_ref, bc2_ref, g2_ref, be2_ref, o_ref):
    s1 = st_ref[0:1]
    s2 = st_ref[1:2]
    gc = jnp.maximum(st_ref[2:3], 1.0)
    mu = s1 / gc
    var = s2 / gc - mu * mu
    a = g_ref[...] / jnp.sqrt(var + EPS)
    dsh = be_ref[...] - mu * a
    cnt = cnt_ref[...][:, 0:1]
    h = a * (nsum_ref[...] / jnp.maximum(cnt, 1.0)) \
        + dsh * (cnt > 0.0).astype(F32)
    hr = h.reshape(B, KP2, 512)
    hg = jnp.sum(hr, axis=1) / float(KP2)
    hgp = jnp.concatenate([hg, jnp.zeros((6, 512), F32)], axis=0)

    rows = (jax.lax.broadcasted_iota(jnp.int32, (56, 1), 0) < B).astype(F32)
    z = jnp.maximum(_bdot(hgp, wc1_ref[...]) + bc1_ref[...], 0.0)
    z = z * rows
    mu1 = jnp.sum(z, axis=0, keepdims=True) / float(B)
    zc = z - mu1
    var1 = jnp.sum(zc * zc * rows, axis=0, keepdims=True) / float(B)
    z = zc / jnp.sqrt(var1 + EPS) * g1_ref[...] + be1_ref[...]
    z2 = jnp.maximum(_bdot(z, wc2_ref[...]) + bc2_ref[...], 0.0)
    z2m = z2 * rows
    mu2 = jnp.sum(z2m, axis=0, keepdims=True) / float(B)
    zc2 = z2 - mu2
    var2 = jnp.sum(zc2 * zc2 * rows, axis=0, keepdims=True) / float(B)
    o_ref[...] = jax.nn.sigmoid(
        zc2 / jnp.sqrt(var2 + EPS) * g2_ref[...] + be2_ref[...])


def _gspec(shape):
    return pl.BlockSpec((GSTEP,) + shape, lambda b: (b, 0, 0))


def _cspec(shape):
    return pl.BlockSpec(shape, lambda b: (0,) * len(shape))


def kernel(x, pos, batch, W1, b1, g1, be1, pw1, W2, b2, g2, be2, pw2,
           W3, b3, g3, be3, Wc1, bc1, gc1, bec1, Wc2, bc2, gc2, bec2):
    f32 = jnp.float32
    xg = jnp.pad(x, ((0, 0), (0, 11))).reshape(B, NG, 64)
    pg = jnp.pad(pos, ((0, 0), (0, 5))).reshape(B, NG, 8)
    wcat1 = jnp.concatenate([
        jnp.pad(W1[:53], ((0, 11), (0, 0))),
        jnp.pad(W1[53:], ((0, 11), (0, 0)))], axis=0)

    r2 = lambda v: v.reshape(1, -1).astype(f32)
    pcol = lambda v: jnp.pad(v.reshape(-1, 1), ((0, 0), (0, 127))).astype(f32)

    nbr1, nsum1, st1 = pl.pallas_call(
        _k1,
        grid=(B // GSTEP,),
        in_specs=[_gspec((NG, 8)), _gspec((NG, 64)),
                  _cspec((128, 128)), _cspec((1, 128))],
        out_specs=[_gspec((NG, 8)), _gspec((NG, 128)), _cspec((8, 128))],
        out_shape=[jax.ShapeDtypeStruct((B, NG, 8), jnp.int32),
                   jax.ShapeDtypeStruct((B, NG, 128), f32),
                   jax.ShapeDtypeStruct((8, 128), f32)],
        interpret=INTERPRET,
    )(pg, xg, wcat1, r2(b1))

    cnt1 = jnp.full((B, NG, 8), 4.0, f32)
    m1 = jnp.ones((B, NG, 8), f32)

    body2 = _mk_pool_conv(NG, KP1, 128, 256)
    nsum2, cnt2, nbr2, m2, st2 = pl.pallas_call(
        body2,
        grid=(B // GSTEP,),
        in_specs=[_gspec((NG, 128)), _gspec((NG, 8)), _gspec((NG, 8)),
                  _gspec((NG, 8)), _cspec((8, 128)), _cspec((1, 128)),
                  _cspec((1, 128)), _cspec((128, 128)),
                  _cspec((256, 256)), _cspec((1, 256))],
        out_specs=[_gspec((KP1, 256)), _gspec((KP1, 8)), _gspec((KP1, 8)),
                   _gspec((KP1, 8)), _cspec((8, 256))],
        out_shape=[jax.ShapeDtypeStruct((B, KP1, 256), f32),
                   jax.ShapeDtypeStruct((B, KP1, 8), f32),
                   jax.ShapeDtypeStruct((B, KP1, 8), jnp.int32),
                   jax.ShapeDtypeStruct((B, KP1, 8), f32),
                   jax.ShapeDtypeStruct((8, 256), f32)],
        interpret=INTERPRET,
    )(nsum1, cnt1, nbr1, m1, st1, r2(g1), r2(be1), pcol(pw1), W2, r2(b2))

    body3 = _mk_pool_conv(KP1, KP2, 256, 512)
    nsum3, cnt3, nbr3, m3, st3 = pl.pallas_call(
        body3,
        grid=(B // GSTEP,),
        in_specs=[_gspec((KP1, 256)), _gspec((KP1, 8)), _gspec((KP1, 8)),
                  _gspec((KP1, 8)), _cspec((8, 256)), _cspec((1, 256)),
                  _cspec((1, 256)), _cspec((256, 128)),
                  _cspec((512, 512)), _cspec((1, 512))],
        out_specs=[_gspec((KP2, 512)), _gspec((KP2, 8)), _gspec((KP2, 8)),
                   _gspec((KP2, 8)), _cspec((8, 512))],
        out_shape=[jax.ShapeDtypeStruct((B, KP2, 512), f32),
                   jax.ShapeDtypeStruct((B, KP2, 8), f32),
                   jax.ShapeDtypeStruct((B, KP2, 8), jnp.int32),
                   jax.ShapeDtypeStruct((B, KP2, 8), f32),
                   jax.ShapeDtypeStruct((8, 512), f32)],
        interpret=INTERPRET,
    )(nsum2, cnt2, nbr2, m2, st2, r2(g2), r2(be2), pcol(pw2), W3, r2(b3))
    del nbr3, m3

    wc2p = jnp.pad(Wc2, ((0, 0), (0, 127)))
    pad1 = lambda v: jnp.pad(v.reshape(1, -1), ((0, 0), (0, 127))).astype(f32)
    out = pl.pallas_call(
        _k45,
        in_specs=[pl.BlockSpec((B * KP2, 512), lambda: (0, 0)),
                  pl.BlockSpec((B * KP2, 8), lambda: (0, 0)),
                  pl.BlockSpec((8, 512), lambda: (0, 0)),
                  pl.BlockSpec((1, 512), lambda: (0, 0)),
                  pl.BlockSpec((1, 512), lambda: (0, 0)),
                  pl.BlockSpec((512, 512), lambda: (0, 0)),
                  pl.BlockSpec((1, 512), lambda: (0, 0)),
                  pl.BlockSpec((1, 512), lambda: (0, 0)),
                  pl.BlockSpec((1, 512), lambda: (0, 0)),
                  pl.BlockSpec((512, 128), lambda: (0, 0)),
                  pl.BlockSpec((1, 128), lambda: (0, 0)),
                  pl.BlockSpec((1, 128), lambda: (0, 0)),
                  pl.BlockSpec((1, 128), lambda: (0, 0))],
        out_specs=pl.BlockSpec((56, 128), lambda: (0, 0)),
        out_shape=jax.ShapeDtypeStruct((56, 128), f32),
        interpret=INTERPRET,
    )(nsum3.reshape(B * KP2, 512), cnt3.reshape(B * KP2, 8), st3,
      r2(g3), r2(be3), Wc1, r2(bc1), r2(gc1), r2(bec1), wc2p, pad1(bc2),
      pad1(gc2), pad1(bec2))

    return out[:B, 0]
